# baseline (speedup 1.0000x reference)
import functools

import jax
import jax.numpy as jnp
from jax import lax
from jax.experimental import pallas as pl
from jax.experimental.pallas import tpu as pltpu
from jax.experimental.pallas import tpu_sc as plsc

N_NODES = 10000
N_EDGES = 320000
D = 128
STEPS = 5
OUT_SIZE = 3

BE = 3200
BN = 2000

_F32 = jnp.float32


def _dot(a, b):
    return jnp.dot(a, b, preferred_element_type=_F32)


def _ln(o, g, be):
    m = jnp.mean(o, axis=-1, keepdims=True)
    v = jnp.mean((o - m) ** 2, axis=-1, keepdims=True)
    return (o - m) / jnp.sqrt(v + 1e-5) * g + be


def _enc_node_body(x_ref, w_ref, aux_ref, o_ref):
    x = x_ref[...]
    h = jnp.maximum(_dot(x, w_ref[0]) + aux_ref[0], 0.0)
    h = jnp.maximum(_dot(h, w_ref[1]) + aux_ref[1], 0.0)
    o_ref[...] = _dot(h, w_ref[2]) + aux_ref[2]


def _enc_edge_body(ea_ref, ew1_ref, ew_ref, eaux_ref, o_ref):
    h = jnp.maximum(_dot(ea_ref[...], ew1_ref[...]) + eaux_ref[0], 0.0)
    h = jnp.maximum(_dot(h, ew_ref[0]) + eaux_ref[1], 0.0)
    o_ref[...] = _dot(h, ew_ref[1]) + eaux_ref[2]


def _proj_body(x_ref, wpre_ref, pa_ref, pb_ref):
    x = x_ref[...]
    pa_ref[...] = _dot(x, wpre_ref[0])
    pb_ref[...] = _dot(x, wpre_ref[1])


def _edge_body(ga_ref, gb_ref, e_ref, w_ref, aux_ref, o_ref):
    h = (ga_ref[...] + gb_ref[...]) + _dot(e_ref[...], w_ref[0])
    h = jnp.maximum(h + aux_ref[0], 0.0)
    h = jnp.maximum(_dot(h, w_ref[1]) + aux_ref[1], 0.0)
    o_ref[...] = _dot(h, w_ref[2]) + aux_ref[2]


def _node_body(x_ref, agg_ref, w_ref, aux_ref, o_ref):
    h = _dot(x_ref[...], w_ref[0]) + _dot(agg_ref[...], w_ref[1])
    h = jnp.maximum(h + aux_ref[0], 0.0)
    h = jnp.maximum(_dot(h, w_ref[2]) + aux_ref[1], 0.0)
    o_ref[...] = _dot(h, w_ref[3]) + aux_ref[2]


def _dec_body(x_ref, wd_ref, daux_ref, out_ref):
    h = jnp.maximum(_dot(x_ref[...], wd_ref[0]) + daux_ref[0], 0.0)
    h = jnp.maximum(_dot(h, wd_ref[1]) + daux_ref[1], 0.0)
    out_ref[...] = _dot(h, wd_ref[2]) + daux_ref[2]


def _full3(a):
    return pl.BlockSpec(a.shape, lambda i: (0,) * a.ndim)


def _rows(block_rows, ncols):
    return pl.BlockSpec((block_rows, ncols), lambda i: (i, 0))


def _enc_node(x, w, aux):
    return pl.pallas_call(
        _enc_node_body,
        grid=(N_NODES // BN,),
        in_specs=[_rows(BN, D), _full3(w), _full3(aux)],
        out_specs=_rows(BN, D),
        out_shape=jax.ShapeDtypeStruct((N_NODES, D), _F32),
    )(x, w, aux)


def _enc_edge(ea, ew1, ew, eaux):
    return pl.pallas_call(
        _enc_edge_body,
        grid=(N_EDGES // BE,),
        in_specs=[_rows(BE, ea.shape[1]), _full3(ew1), _full3(ew), _full3(eaux)],
        out_specs=_rows(BE, D),
        out_shape=jax.ShapeDtypeStruct((N_EDGES, D), _F32),
    )(ea, ew1, ew, eaux)


def _proj(x, wpre):
    out = jax.ShapeDtypeStruct((N_NODES, D), _F32)
    return pl.pallas_call(
        _proj_body,
        grid=(N_NODES // BN,),
        in_specs=[_rows(BN, D), _full3(wpre)],
        out_specs=[_rows(BN, D)] * 2,
        out_shape=[out, out],
    )(x, wpre)


def _edge(ga, gb, e, w, aux):
    return pl.pallas_call(
        _edge_body,
        grid=(N_EDGES // BE,),
        in_specs=[_rows(BE, D), _rows(BE, D), _rows(BE, D), _full3(w), _full3(aux)],
        out_specs=_rows(BE, D),
        out_shape=jax.ShapeDtypeStruct((N_EDGES, D), _F32),
    )(ga, gb, e, w, aux)


def _node(x, agg, w, aux):
    return pl.pallas_call(
        _node_body,
        grid=(N_NODES // BN,),
        in_specs=[_rows(BN, D), _rows(BN, D), _full3(w), _full3(aux)],
        out_specs=_rows(BN, D),
        out_shape=jax.ShapeDtypeStruct((N_NODES, D), _F32),
    )(x, agg, w, aux)


def _dec(x, wd, daux):
    return pl.pallas_call(
        _dec_body,
        grid=(N_NODES // BN,),
        in_specs=[_rows(BN, D), _full3(wd), _full3(daux)],
        out_specs=_rows(BN, D),
        out_shape=jax.ShapeDtypeStruct((N_NODES, D), _F32),
    )(x, wd, daux)


def _aux(*rows):
    a = jnp.zeros((8, D), _F32)
    for i, r in enumerate(rows):
        a = a.at[i].set(r)
    return a


def _pack(params):
    p = {}
    en = params['enc_node']['mlp']
    pe0 = params['proc'][0]['edge']['mlp']
    p['enc_w'] = jnp.stack([en[0]['W'], en[1]['W'], en[2]['W']])
    p['enc_wpre'] = jnp.stack([pe0[0]['W'][:D], pe0[0]['W'][D:2 * D]])
    p['enc_aux'] = _aux(en[0]['b'], en[1]['b'], en[2]['b'],
                        params['enc_node']['g'], params['enc_node']['be'])
    ee = params['enc_edge']['mlp']
    p['ee_w1'] = ee[0]['W']
    p['ee_w'] = jnp.stack([ee[1]['W'], ee[2]['W']])
    p['ee_aux'] = _aux(ee[0]['b'], ee[1]['b'], ee[2]['b'],
                       params['enc_edge']['g'], params['enc_edge']['be'])
    p['edge'] = []
    p['node'] = []
    for i in range(STEPS):
        pe = params['proc'][i]['edge']
        pn = params['proc'][i]['node']
        p['edge'].append({
            'w': jnp.stack([pe['mlp'][0]['W'][2 * D:], pe['mlp'][1]['W'],
                            pe['mlp'][2]['W']]),
            'aux': _aux(pe['mlp'][0]['b'], pe['mlp'][1]['b'], pe['mlp'][2]['b'],
                        pe['g'], pe['be']),
        })
        nd = {
            'w': jnp.stack([pn['mlp'][0]['W'][:D], pn['mlp'][0]['W'][D:],
                            pn['mlp'][1]['W'], pn['mlp'][2]['W']]),
            'aux': _aux(pn['mlp'][0]['b'], pn['mlp'][1]['b'], pn['mlp'][2]['b'],
                        pn['g'], pn['be']),
        }
        if i + 1 < STEPS:
            pe1 = params['proc'][i + 1]['edge']['mlp']
            nd['wpre'] = jnp.stack([pe1[0]['W'][:D], pe1[0]['W'][D:2 * D]])
        p['node'].append(nd)
    dc = params['dec']
    w3 = jnp.zeros((D, D), _F32).at[:, :OUT_SIZE].set(dc[2]['W'])
    b3 = jnp.zeros((D,), _F32).at[:OUT_SIZE].set(dc[2]['b'])
    p['dec_w'] = jnp.stack([dc[0]['W'], dc[1]['W'], w3])
    p['dec_aux'] = _aux(dc[0]['b'], dc[1]['b'], b3)
    return p


_NC = 2
_NS = 16
_NW = _NC * _NS
_EPW = N_EDGES // _NW
_CH = 80
_NCH = _EPW // _CH

_sc_mesh = plsc.VectorSubcoreMesh(core_axis_name="c", subcore_axis_name="s")


@functools.partial(
    pl.kernel, mesh=_sc_mesh,
    out_type=[jax.ShapeDtypeStruct((N_EDGES, D), _F32),
              jax.ShapeDtypeStruct((N_EDGES, D), _F32)],
    scratch_types=[pltpu.VMEM((2, _CH), jnp.int32), pltpu.VMEM((2, _CH), jnp.int32),
                   pltpu.VMEM((2, _CH, D), _F32), pltpu.VMEM((2, _CH, D), _F32)]
                  + [pltpu.SemaphoreType.DMA] * 6,
)
def _sc_gather(pa_hbm, pb_hbm, s_hbm, r_hbm, ga_hbm, gb_hbm,
               sidx, ridx, bufa, bufb, si0, si1, sg0, sg1, sw0, sw1):
    wid = lax.axis_index("s") * _NC + lax.axis_index("c")
    base = wid * _EPW
    semi, semg, semw = (si0, si1), (sg0, sg1), (sw0, sw1)

    def idx_load(ch, b):
        off = base + ch * _CH
        pltpu.async_copy(s_hbm.at[pl.ds(off, _CH)], sidx.at[b], semi[b])
        pltpu.async_copy(r_hbm.at[pl.ds(off, _CH)], ridx.at[b], semi[b])

    def idx_wait(b):
        pltpu.make_async_copy(s_hbm.at[pl.ds(0, _CH)], sidx.at[b], semi[b]).wait()
        pltpu.make_async_copy(r_hbm.at[pl.ds(0, _CH)], ridx.at[b], semi[b]).wait()

    def gath(b):
        pltpu.async_copy(pa_hbm.at[sidx.at[b]], bufa.at[b], semg[b])
        pltpu.async_copy(pb_hbm.at[ridx.at[b]], bufb.at[b], semg[b])

    def gath_wait(b):
        pltpu.make_async_copy(pa_hbm.at[sidx.at[b]], bufa.at[b], semg[b]).wait()
        pltpu.make_async_copy(pb_hbm.at[ridx.at[b]], bufb.at[b], semg[b]).wait()

    def wr(ch, b):
        off = base + ch * _CH
        pltpu.async_copy(bufa.at[b], ga_hbm.at[pl.ds(off, _CH)], semw[b])
        pltpu.async_copy(bufb.at[b], gb_hbm.at[pl.ds(off, _CH)], semw[b])

    def wr_wait(b):
        pltpu.make_async_copy(bufa.at[b], ga_hbm.at[pl.ds(0, _CH)], semw[b]).wait()
        pltpu.make_async_copy(bufb.at[b], gb_hbm.at[pl.ds(0, _CH)], semw[b]).wait()

    idx_load(0, 0)
    idx_load(1, 1)
    idx_wait(0)
    gath(0)

    def pair(p, carry):
        i0 = 2 * p + 1
        @pl.when(p >= 1)
        def _():
            wr_wait(1)
        idx_wait(1)
        gath(1)
        gath_wait(0)
        idx_load(i0 + 1, 0)
        wr(i0 - 1, 0)
        i1 = i0 + 1
        wr_wait(0)
        idx_wait(0)
        gath(0)
        gath_wait(1)
        @pl.when(p < (_NCH - 3) // 2)
        def _():
            idx_load(i1 + 1, 1)
        wr(i1 - 1, 1)
        return carry

    lax.fori_loop(0, (_NCH - 1) // 2, pair, 0)
    gath_wait(0)
    wr(_NCH - 1, 0)
    wr_wait(1)
    wr_wait(0)


def _gather(pa, pb, s, r):
    return _sc_gather(pa, pb, s, r)


_NP = 10240
_NPS = _NP // _NW
_EPAD = N_EDGES + 96


_RPS = _NP // _NS
_ZR = 128


@functools.partial(
    pl.kernel, mesh=_sc_mesh,
    out_type=jax.ShapeDtypeStruct((_NC, _NP, D), _F32),
    scratch_types=[pltpu.VMEM((2, _CH), jnp.int32), pltpu.VMEM((2, _CH, D), _F32),
                   pltpu.VMEM_SHARED((_NP, D), _F32)]
                  + [pltpu.SemaphoreType.DMA] * 4,
)
def _sc_scatter(ue_hbm, r_hbm, z_hbm, out_hbm, ridx, buf, acc_sh,
                si0, si1, ss0, ss1):
    cid = lax.axis_index("c")
    sid = lax.axis_index("s")
    wid = sid * _NC + cid
    rbase = sid * _RPS
    semi, sems = (si0, si1), (ss0, ss1)

    def zs(i, c):
        pltpu.sync_copy(z_hbm, acc_sh.at[pl.ds(rbase + i * _ZR, _ZR)])
        return c

    lax.fori_loop(0, _RPS // _ZR, zs, 0)
    plsc.subcore_barrier()
    base = wid * _EPW

    def ld(ch, b):
        off = base + ch * _CH
        pltpu.async_copy(r_hbm.at[pl.ds(off, _CH)], ridx.at[b], semi[b])
        pltpu.async_copy(ue_hbm.at[pl.ds(off, _CH)], buf.at[b], semi[b])

    def ld_wait(b):
        pltpu.make_async_copy(r_hbm.at[pl.ds(0, _CH)], ridx.at[b], semi[b]).wait()
        pltpu.make_async_copy(ue_hbm.at[pl.ds(0, _CH)], buf.at[b], semi[b]).wait()

    def sca(b):
        pltpu.async_copy(buf.at[b], acc_sh.at[ridx.at[b]], sems[b], add=True)

    def sca_wait(b):
        pltpu.make_async_copy(buf.at[b], acc_sh.at[ridx.at[b]], sems[b]).wait()

    ld(0, 0)
    ld(1, 1)
    ld_wait(0)
    sca(0)

    def pair(p, carry):
        i0 = 2 * p + 1
        ld_wait(1)
        sca(1)
        sca_wait(0)
        ld(i0 + 1, 0)
        i1 = i0 + 1
        ld_wait(0)
        sca(0)
        sca_wait(1)
        @pl.when(p < (_NCH - 3) // 2)
        def _():
            ld(i1 + 1, 1)
        return carry

    lax.fori_loop(0, (_NCH - 1) // 2, pair, 0)
    sca_wait(0)
    plsc.subcore_barrier()
    pltpu.sync_copy(acc_sh.at[pl.ds(rbase, _RPS)],
                    out_hbm.at[cid, pl.ds(rbase, _RPS)])


def _segment_sum(ue, r):
    z = jnp.zeros((_ZR, D), _F32)
    parts = _sc_scatter(ue, r, z)
    return parts[0, :N_NODES] + parts[1, :N_NODES]


def kernel(x, edge_attr, params, edge_index):
    p = _pack(params)
    s = edge_index[0]
    r = edge_index[1]
    xc = _ln(_enc_node(x, p['enc_w'], p['enc_aux']),
             params['enc_node']['g'], params['enc_node']['be'])
    e = _ln(_enc_edge(edge_attr, p['ee_w1'], p['ee_w'], p['ee_aux']),
            params['enc_edge']['g'], params['enc_edge']['be'])
    pa, pb = _proj(xc, p['enc_wpre'])
    for i in range(STEPS):
        ga, gb = _gather(pa, pb, s, r)
        pe = params['proc'][i]['edge']
        ue = _ln(_edge(ga, gb, e, p['edge'][i]['w'], p['edge'][i]['aux']),
                 pe['g'], pe['be'])
        agg = _segment_sum(ue, r)
        pn = params['proc'][i]['node']
        xc = _ln(_node(xc, agg, p['node'][i]['w'], p['node'][i]['aux']),
                 pn['g'], pn['be'])
        if i + 1 < STEPS:
            pa, pb = _proj(xc, p['node'][i]['wpre'])
            e = ue
    out = _dec(xc, p['dec_w'], p['dec_aux'])
    return out[:, :OUT_SIZE]


# speedup vs baseline: 1.7337x; 1.7337x over previous
import functools

import jax
import jax.numpy as jnp
from jax import lax
from jax.experimental import pallas as pl
from jax.experimental.pallas import tpu as pltpu
from jax.experimental.pallas import tpu_sc as plsc

N_NODES = 10000
N_EDGES = 320000
D = 128
STEPS = 5
OUT_SIZE = 3

BE = 3200
BN = 2000

_F32 = jnp.float32


def _dot(a, b):
    return jnp.dot(a, b, preferred_element_type=_F32)


def _ln(o, g, be):
    m = jnp.mean(o, axis=-1, keepdims=True)
    v = jnp.mean((o - m) ** 2, axis=-1, keepdims=True)
    return (o - m) / jnp.sqrt(v + 1e-5) * g + be


def _enc_node_body(x_ref, w_ref, aux_ref, o_ref):
    x = x_ref[...]
    h = jnp.maximum(_dot(x, w_ref[0]) + aux_ref[0], 0.0)
    h = jnp.maximum(_dot(h, w_ref[1]) + aux_ref[1], 0.0)
    o = _dot(h, w_ref[2]) + aux_ref[2]
    o_ref[...] = _ln(o, aux_ref[3], aux_ref[4])


def _enc_edge_body(ea_ref, ew1_ref, ew_ref, eaux_ref, o_ref):
    h = jnp.maximum(_dot(ea_ref[...], ew1_ref[...]) + eaux_ref[0], 0.0)
    h = jnp.maximum(_dot(h, ew_ref[0]) + eaux_ref[1], 0.0)
    o = _dot(h, ew_ref[1]) + eaux_ref[2]
    o_ref[...] = _ln(o, eaux_ref[3], eaux_ref[4])


def _proj_body(x_ref, wpre_ref, pa_ref, pb_ref):
    x = x_ref[...]
    pa_ref[...] = _dot(x, wpre_ref[0])
    pb_ref[...] = _dot(x, wpre_ref[1])


def _edge_body(ga_ref, gb_ref, e_ref, w_ref, aux_ref, o_ref):
    h = (ga_ref[...] + gb_ref[...]) + _dot(e_ref[...], w_ref[0])
    h = jnp.maximum(h + aux_ref[0], 0.0)
    h = jnp.maximum(_dot(h, w_ref[1]) + aux_ref[1], 0.0)
    o = _dot(h, w_ref[2]) + aux_ref[2]
    o_ref[...] = _ln(o, aux_ref[3], aux_ref[4])


def _node_body(x_ref, agg_ref, w_ref, aux_ref, o_ref):
    h = _dot(x_ref[...], w_ref[0]) + _dot(agg_ref[...], w_ref[1])
    h = jnp.maximum(h + aux_ref[0], 0.0)
    h = jnp.maximum(_dot(h, w_ref[2]) + aux_ref[1], 0.0)
    o = _dot(h, w_ref[3]) + aux_ref[2]
    o_ref[...] = _ln(o, aux_ref[3], aux_ref[4])


def _dec_body(x_ref, wd_ref, daux_ref, out_ref):
    h = jnp.maximum(_dot(x_ref[...], wd_ref[0]) + daux_ref[0], 0.0)
    h = jnp.maximum(_dot(h, wd_ref[1]) + daux_ref[1], 0.0)
    out_ref[...] = _dot(h, wd_ref[2]) + daux_ref[2]


def _full3(a):
    return pl.BlockSpec(a.shape, lambda i: (0,) * a.ndim)


def _rows(block_rows, ncols):
    return pl.BlockSpec((block_rows, ncols), lambda i: (i, 0))


def _enc_node(x, w, aux):
    return pl.pallas_call(
        _enc_node_body,
        grid=(N_NODES // BN,),
        in_specs=[_rows(BN, D), _full3(w), _full3(aux)],
        out_specs=_rows(BN, D),
        out_shape=jax.ShapeDtypeStruct((N_NODES, D), _F32),
    )(x, w, aux)


def _enc_edge(ea, ew1, ew, eaux):
    return pl.pallas_call(
        _enc_edge_body,
        grid=(N_EDGES // BE,),
        in_specs=[_rows(BE, ea.shape[1]), _full3(ew1), _full3(ew), _full3(eaux)],
        out_specs=_rows(BE, D),
        out_shape=jax.ShapeDtypeStruct((N_EDGES, D), _F32),
    )(ea, ew1, ew, eaux)


def _proj(x, wpre):
    out = jax.ShapeDtypeStruct((N_NODES, D), _F32)
    return pl.pallas_call(
        _proj_body,
        grid=(N_NODES // BN,),
        in_specs=[_rows(BN, D), _full3(wpre)],
        out_specs=[_rows(BN, D)] * 2,
        out_shape=[out, out],
    )(x, wpre)


def _edge(ga, gb, e, w, aux):
    return pl.pallas_call(
        _edge_body,
        grid=(N_EDGES // BE,),
        in_specs=[_rows(BE, D), _rows(BE, D), _rows(BE, D), _full3(w), _full3(aux)],
        out_specs=_rows(BE, D),
        out_shape=jax.ShapeDtypeStruct((N_EDGES, D), _F32),
    )(ga, gb, e, w, aux)


def _node(x, agg, w, aux):
    return pl.pallas_call(
        _node_body,
        grid=(N_NODES // BN,),
        in_specs=[_rows(BN, D), _rows(BN, D), _full3(w), _full3(aux)],
        out_specs=_rows(BN, D),
        out_shape=jax.ShapeDtypeStruct((N_NODES, D), _F32),
    )(x, agg, w, aux)


def _dec(x, wd, daux):
    return pl.pallas_call(
        _dec_body,
        grid=(N_NODES // BN,),
        in_specs=[_rows(BN, D), _full3(wd), _full3(daux)],
        out_specs=_rows(BN, D),
        out_shape=jax.ShapeDtypeStruct((N_NODES, D), _F32),
    )(x, wd, daux)


def _aux(*rows):
    a = jnp.zeros((8, D), _F32)
    for i, r in enumerate(rows):
        a = a.at[i].set(r)
    return a


def _pack(params):
    p = {}
    en = params['enc_node']['mlp']
    pe0 = params['proc'][0]['edge']['mlp']
    p['enc_w'] = jnp.stack([en[0]['W'], en[1]['W'], en[2]['W']])
    p['enc_wpre'] = jnp.stack([pe0[0]['W'][:D], pe0[0]['W'][D:2 * D]])
    p['enc_aux'] = _aux(en[0]['b'], en[1]['b'], en[2]['b'],
                        params['enc_node']['g'], params['enc_node']['be'])
    ee = params['enc_edge']['mlp']
    p['ee_w1'] = ee[0]['W']
    p['ee_w'] = jnp.stack([ee[1]['W'], ee[2]['W']])
    p['ee_aux'] = _aux(ee[0]['b'], ee[1]['b'], ee[2]['b'],
                       params['enc_edge']['g'], params['enc_edge']['be'])
    p['edge'] = []
    p['node'] = []
    for i in range(STEPS):
        pe = params['proc'][i]['edge']
        pn = params['proc'][i]['node']
        p['edge'].append({
            'w': jnp.stack([pe['mlp'][0]['W'][2 * D:], pe['mlp'][1]['W'],
                            pe['mlp'][2]['W']]),
            'aux': _aux(pe['mlp'][0]['b'], pe['mlp'][1]['b'], pe['mlp'][2]['b'],
                        pe['g'], pe['be']),
        })
        nd = {
            'w': jnp.stack([pn['mlp'][0]['W'][:D], pn['mlp'][0]['W'][D:],
                            pn['mlp'][1]['W'], pn['mlp'][2]['W']]),
            'aux': _aux(pn['mlp'][0]['b'], pn['mlp'][1]['b'], pn['mlp'][2]['b'],
                        pn['g'], pn['be']),
        }
        if i + 1 < STEPS:
            pe1 = params['proc'][i + 1]['edge']['mlp']
            nd['wpre'] = jnp.stack([pe1[0]['W'][:D], pe1[0]['W'][D:2 * D]])
        p['node'].append(nd)
    dc = params['dec']
    w3 = jnp.zeros((D, D), _F32).at[:, :OUT_SIZE].set(dc[2]['W'])
    b3 = jnp.zeros((D,), _F32).at[:OUT_SIZE].set(dc[2]['b'])
    p['dec_w'] = jnp.stack([dc[0]['W'], dc[1]['W'], w3])
    p['dec_aux'] = _aux(dc[0]['b'], dc[1]['b'], b3)
    return p


_NC = 2
_NS = 16
_NW = _NC * _NS
_EPW = N_EDGES // _NW
_CH = 80
_NCH = _EPW // _CH

_sc_mesh = plsc.VectorSubcoreMesh(core_axis_name="c", subcore_axis_name="s")


@functools.partial(
    pl.kernel, mesh=_sc_mesh,
    out_type=[jax.ShapeDtypeStruct((N_EDGES, D), _F32),
              jax.ShapeDtypeStruct((N_EDGES, D), _F32)],
    scratch_types=[pltpu.VMEM((2, _CH), jnp.int32), pltpu.VMEM((2, _CH), jnp.int32),
                   pltpu.VMEM((2, _CH, D), _F32), pltpu.VMEM((2, _CH, D), _F32)]
                  + [pltpu.SemaphoreType.DMA] * 6,
)
def _sc_gather(pa_hbm, pb_hbm, s_hbm, r_hbm, ga_hbm, gb_hbm,
               sidx, ridx, bufa, bufb, si0, si1, sg0, sg1, sw0, sw1):
    wid = lax.axis_index("s") * _NC + lax.axis_index("c")
    base = wid * _EPW
    semi, semg, semw = (si0, si1), (sg0, sg1), (sw0, sw1)

    def idx_load(ch, b):
        off = base + ch * _CH
        pltpu.async_copy(s_hbm.at[pl.ds(off, _CH)], sidx.at[b], semi[b])
        pltpu.async_copy(r_hbm.at[pl.ds(off, _CH)], ridx.at[b], semi[b])

    def idx_wait(b):
        pltpu.make_async_copy(s_hbm.at[pl.ds(0, _CH)], sidx.at[b], semi[b]).wait()
        pltpu.make_async_copy(r_hbm.at[pl.ds(0, _CH)], ridx.at[b], semi[b]).wait()

    def gath(b):
        pltpu.async_copy(pa_hbm.at[sidx.at[b]], bufa.at[b], semg[b])
        pltpu.async_copy(pb_hbm.at[ridx.at[b]], bufb.at[b], semg[b])

    def gath_wait(b):
        pltpu.make_async_copy(pa_hbm.at[sidx.at[b]], bufa.at[b], semg[b]).wait()
        pltpu.make_async_copy(pb_hbm.at[ridx.at[b]], bufb.at[b], semg[b]).wait()

    def wr(ch, b):
        off = base + ch * _CH
        pltpu.async_copy(bufa.at[b], ga_hbm.at[pl.ds(off, _CH)], semw[b])
        pltpu.async_copy(bufb.at[b], gb_hbm.at[pl.ds(off, _CH)], semw[b])

    def wr_wait(b):
        pltpu.make_async_copy(bufa.at[b], ga_hbm.at[pl.ds(0, _CH)], semw[b]).wait()
        pltpu.make_async_copy(bufb.at[b], gb_hbm.at[pl.ds(0, _CH)], semw[b]).wait()

    idx_load(0, 0)
    idx_load(1, 1)
    idx_wait(0)
    gath(0)

    def pair(p, carry):
        i0 = 2 * p + 1
        @pl.when(p >= 1)
        def _():
            wr_wait(1)
        idx_wait(1)
        gath(1)
        gath_wait(0)
        idx_load(i0 + 1, 0)
        wr(i0 - 1, 0)
        i1 = i0 + 1
        wr_wait(0)
        idx_wait(0)
        gath(0)
        gath_wait(1)
        @pl.when(p < (_NCH - 3) // 2)
        def _():
            idx_load(i1 + 1, 1)
        wr(i1 - 1, 1)
        return carry

    lax.fori_loop(0, (_NCH - 1) // 2, pair, 0)
    gath_wait(0)
    wr(_NCH - 1, 0)
    wr_wait(1)
    wr_wait(0)


def _gather(pa, pb, s, r):
    return _sc_gather(pa, pb, s, r)


_NP = 10240
_NPS = _NP // _NW
_EPAD = N_EDGES + 96


_RPS = _NP // _NS
_ZR = 128


@functools.partial(
    pl.kernel, mesh=_sc_mesh,
    out_type=jax.ShapeDtypeStruct((_NC, _NP, D), _F32),
    scratch_types=[pltpu.VMEM((2, _CH), jnp.int32), pltpu.VMEM((2, _CH, D), _F32),
                   pltpu.VMEM_SHARED((_NP, D), _F32)]
                  + [pltpu.SemaphoreType.DMA] * 4,
)
def _sc_scatter(ue_hbm, r_hbm, z_hbm, out_hbm, ridx, buf, acc_sh,
                si0, si1, ss0, ss1):
    cid = lax.axis_index("c")
    sid = lax.axis_index("s")
    wid = sid * _NC + cid
    rbase = sid * _RPS
    semi, sems = (si0, si1), (ss0, ss1)

    def zs(i, c):
        pltpu.sync_copy(z_hbm, acc_sh.at[pl.ds(rbase + i * _ZR, _ZR)])
        return c

    lax.fori_loop(0, _RPS // _ZR, zs, 0)
    plsc.subcore_barrier()
    base = wid * _EPW

    def ld(ch, b):
        off = base + ch * _CH
        pltpu.async_copy(r_hbm.at[pl.ds(off, _CH)], ridx.at[b], semi[b])
        pltpu.async_copy(ue_hbm.at[pl.ds(off, _CH)], buf.at[b], semi[b])

    def ld_wait(b):
        pltpu.make_async_copy(r_hbm.at[pl.ds(0, _CH)], ridx.at[b], semi[b]).wait()
        pltpu.make_async_copy(ue_hbm.at[pl.ds(0, _CH)], buf.at[b], semi[b]).wait()

    def sca(b):
        pltpu.async_copy(buf.at[b], acc_sh.at[ridx.at[b]], sems[b], add=True)

    def sca_wait(b):
        pltpu.make_async_copy(buf.at[b], acc_sh.at[ridx.at[b]], sems[b]).wait()

    ld(0, 0)
    ld(1, 1)
    ld_wait(0)
    sca(0)

    def pair(p, carry):
        i0 = 2 * p + 1
        ld_wait(1)
        sca(1)
        sca_wait(0)
        ld(i0 + 1, 0)
        i1 = i0 + 1
        ld_wait(0)
        sca(0)
        sca_wait(1)
        @pl.when(p < (_NCH - 3) // 2)
        def _():
            ld(i1 + 1, 1)
        return carry

    lax.fori_loop(0, (_NCH - 1) // 2, pair, 0)
    sca_wait(0)
    plsc.subcore_barrier()
    pltpu.sync_copy(acc_sh.at[pl.ds(rbase, _RPS)],
                    out_hbm.at[cid, pl.ds(rbase, _RPS)])


def _segment_sum(ue, r):
    z = jnp.zeros((_ZR, D), _F32)
    parts = _sc_scatter(ue, r, z)
    return parts[0, :N_NODES] + parts[1, :N_NODES]


def kernel(x, edge_attr, params, edge_index):
    p = _pack(params)
    s = edge_index[0]
    r = edge_index[1]
    xc = _enc_node(x, p['enc_w'], p['enc_aux'])
    e = _enc_edge(edge_attr, p['ee_w1'], p['ee_w'], p['ee_aux'])
    pa, pb = _proj(xc, p['enc_wpre'])
    for i in range(STEPS):
        ga, gb = _gather(pa, pb, s, r)
        ue = _edge(ga, gb, e, p['edge'][i]['w'], p['edge'][i]['aux'])
        agg = _segment_sum(ue, r)
        xc = _node(xc, agg, p['node'][i]['w'], p['node'][i]['aux'])
        if i + 1 < STEPS:
            pa, pb = _proj(xc, p['node'][i]['wpre'])
            e = ue
    out = _dec(xc, p['dec_w'], p['dec_aux'])
    return out[:, :OUT_SIZE]
